# Initial kernel scaffold; baseline (speedup 1.0000x reference)
import jax
import jax.numpy as jnp
from jax.experimental import pallas as pl


def kernel(ids_per_cls_train, budget, feats, reps, d):
    raise NotImplementedError("write your pallas kernel here")


# speedup vs baseline: 14.6331x; 14.6331x over previous
import functools

import numpy as np
import jax
import jax.numpy as jnp
from jax import lax
from jax.experimental import pallas as pl
from jax.experimental.pallas import tpu as pltpu
from jax.experimental.pallas import tpu_sc as plsc

_N_CLS = 20
_PER = 200
_SEL = 100
_DIM = 64
_NPAD = 4096
_NC = 2
_NS = 16
_NW = _NC * _NS
_BPW = _NPAD // _NW


_M32 = np.uint64(0xFFFFFFFF)


def _tf2x32(k0, k1, x0, x1):
    k0 = np.asarray(k0).astype(np.uint64)
    k1 = np.asarray(k1).astype(np.uint64)
    x0 = np.asarray(x0).astype(np.uint64)
    x1 = np.asarray(x1).astype(np.uint64)
    ks2 = (k0 ^ k1 ^ np.uint64(0x1BD11BDA)) & _M32
    ks = (k0, k1, ks2)
    rots = ((13, 15, 26, 6), (17, 29, 16, 24))
    x0 = (x0 + k0) & _M32
    x1 = (x1 + k1) & _M32
    for i in range(5):
        for r in rots[i % 2]:
            x0 = (x0 + x1) & _M32
            x1 = (((x1 << np.uint64(r)) | (x1 >> np.uint64(32 - r))) & _M32) ^ x0
        x0 = (x0 + ks[(i + 1) % 3]) & _M32
        x1 = (x1 + ks[(i + 2) % 3] + np.uint64(i + 1)) & _M32
    return x0.astype(np.uint32), x1.astype(np.uint32)


def _np_randint(key, n, span):
    b1, b2 = _tf2x32(key[0], key[1], np.zeros(2, np.uint32),
                     np.arange(2, dtype=np.uint32))
    k1, k2 = (b1[0], b2[0]), (b1[1], b2[1])
    zeros = np.zeros(n, np.uint32)
    cnt = np.arange(n, dtype=np.uint32)
    hi = (lambda a, b: a ^ b)(*_tf2x32(k1[0], k1[1], zeros, cnt)).astype(np.uint64)
    lo = (lambda a, b: a ^ b)(*_tf2x32(k2[0], k2[1], zeros, cnt)).astype(np.uint64)
    span = np.uint64(span)
    mult = np.uint64(2 ** 16) % span
    mult = (mult * mult) % span
    return (((hi % span) * mult + (lo % span)) % span).astype(np.int64)


def _build_weights() -> np.ndarray:
    w = np.zeros((_N_CLS, _NPAD), np.float32)
    for i in range(_N_CLS):
        for j in range(_N_CLS):
            if j == i:
                continue
            fk0, fk1 = _tf2x32(0, 1, 0, i * _N_CLS + j)
            pick = _np_randint((fk0, fk1), _PER, _PER)
            np.add.at(w[i], j * _PER + pick, 1.0)
    return w.reshape(_N_CLS, 1, _NPAD)


_W = _build_weights()


def _sc_gather(table, idx):
    mesh = plsc.VectorSubcoreMesh(core_axis_name="c", subcore_axis_name="s")

    @functools.partial(
        pl.kernel,
        mesh=mesh,
        out_type=jax.ShapeDtypeStruct((_NPAD, 2 * _DIM), jnp.float32),
        scratch_types=[
            pltpu.VMEM((_BPW,), jnp.int32),
            pltpu.VMEM((_BPW, 2 * _DIM), jnp.float32),
            pltpu.SemaphoreType.DMA,
        ],
    )
    def gk(table_hbm, idx_hbm, out_hbm, idx_v, rows_v, sem):
        wid = lax.axis_index("s") * _NC + lax.axis_index("c")
        base = wid * _BPW
        pltpu.sync_copy(idx_hbm.at[pl.ds(base, _BPW)], idx_v)
        pltpu.async_copy(table_hbm.at[idx_v], rows_v, sem).wait()
        pltpu.sync_copy(rows_v, out_hbm.at[pl.ds(base, _BPW)])

    return gk(table, idx)


def _f16_quant(x):
    bits = lax.bitcast_convert_type(x, jnp.int32)
    absb = bits & 0x7FFFFFFF
    signb = bits ^ absb
    lsb = (absb >> 13) & 1
    rounded = (absb + 4095 + lsb) & jnp.int32(-8192)
    ynorm = lax.bitcast_convert_type(signb | rounded, jnp.float32)
    s = x * 16777216.0
    r = (s + 12582912.0) - 12582912.0
    ysub = r * 5.9604644775390625e-08
    return jnp.where(absb < (113 << 23), ysub, ynorm)


def _f16_lt_threshold(d32):
    db = lax.bitcast_convert_type(d32, jnp.int32)
    lsb = (db >> 13) & 1
    hb = ((db + 4095 + lsb) >> 13) - (112 << 10)
    pe, pm = hb >> 10, hb & 1023
    d16v = lax.bitcast_convert_type(((pe + 112) << 23) | (pm << 13),
                                    jnp.float32)
    pb = hb - 1
    qe, qm = pb >> 10, pb & 1023
    pred = lax.bitcast_convert_type(((qe + 112) << 23) | (qm << 13),
                                    jnp.float32)
    mid = 0.5 * (pred + d16v)
    midb = lax.bitcast_convert_type(mid, jnp.int32)
    return lax.bitcast_convert_type(midb + (hb & 1), jnp.float32)


def _row_of(col):
    return lax.dot_general(
        jnp.ones((1, 1), jnp.float32), col,
        (((1,), (1,)), ((), ())),
        precision=lax.Precision.HIGHEST,
        preferred_element_type=jnp.float32,
    )


def _tc_body(d_ref, start_ref, g_ref, par_ref, w_ref, ids_ref, out_ref,
             g16_ref, nb_ref):
    i = pl.program_id(0)

    @pl.when(i == 0)
    def _():
        g128 = g_ref[...]
        par = par_ref[...]
        g64 = jnp.where(par == 1, g128[:, _DIM:], g128[:, :_DIM])
        g16_ref[...] = _f16_quant(g64)
        g32_0 = g16_ref[...]
        nb_col = jnp.sum(g32_0 * g32_0, axis=1, keepdims=True)
        nb_ref[...] = _row_of(nb_col)

    g32 = g16_ref[...]
    a32 = g16_ref[pl.ds(i * _PER, _PER), :]
    prod = lax.dot_general(
        a32, g32, (((1,), (1,)), ((), ())),
        precision=lax.Precision.DEFAULT,
        preferred_element_type=jnp.float32,
    )
    na = jnp.sum(a32 * a32, axis=1, keepdims=True)
    sq = (na + nb_ref[...]) - 2.0 * prod
    dist = jnp.sqrt(jnp.maximum(sq, 0.0))
    cmpf = (dist < _f16_lt_threshold(d_ref[0])).astype(jnp.float32)
    wsum = jnp.sum(cmpf * w_ref[0], axis=1, keepdims=True)
    counts = wsum.astype(jnp.int32)

    idx_col = lax.broadcasted_iota(jnp.int32, (_PER, 1), 0)
    key_col = counts * 256 + idx_col
    key_row = _row_of(key_col.astype(jnp.float32)).astype(jnp.int32)
    less = key_row < key_col
    rank = jnp.sum(less.astype(jnp.int32), axis=1, keepdims=True)

    tgt = lax.broadcasted_iota(jnp.int32, (1, _SEL), 1) + start_ref[0]
    hit = rank == tgt
    ids_col = ids_ref[0]
    out_ref[0] = jnp.sum(jnp.where(hit, ids_col, 0), axis=0, keepdims=True)


def _tc_main(g, par, w, ids3, d_arr, start_arr):
    return pl.pallas_call(
        _tc_body,
        grid=(_N_CLS,),
        in_specs=[
            pl.BlockSpec(memory_space=pltpu.SMEM),
            pl.BlockSpec(memory_space=pltpu.SMEM),
            pl.BlockSpec((_NPAD, 2 * _DIM), lambda i: (0, 0)),
            pl.BlockSpec((_NPAD, 1), lambda i: (0, 0)),
            pl.BlockSpec((1, 1, _NPAD), lambda i: (i, 0, 0)),
            pl.BlockSpec((1, _PER, 1), lambda i: (i, 0, 0)),
        ],
        out_specs=pl.BlockSpec((1, 1, _SEL), lambda i: (i, 0, 0)),
        out_shape=jax.ShapeDtypeStruct((_N_CLS, 1, _SEL), jnp.int32),
        scratch_shapes=[
            pltpu.VMEM((_NPAD, _DIM), jnp.float32),
            pltpu.VMEM((1, _NPAD), jnp.float32),
        ],
    )(d_arr, start_arr, g, par, w, ids3)


def kernel(ids_per_cls_train, budget, feats, reps, d):
    ids_flat = ids_per_cls_train.reshape(-1).astype(jnp.int32)
    idx = jnp.concatenate(
        [ids_flat, jnp.zeros((_NPAD - _N_CLS * _PER,), jnp.int32)])
    table2 = reps.reshape(reps.shape[0] // 2, 2 * _DIM)
    g = _sc_gather(table2, idx // 2)
    par = (idx % 2).astype(jnp.int32).reshape(_NPAD, 1)
    ids3 = ids_per_cls_train.reshape(_N_CLS, _PER, 1).astype(jnp.int32)
    d_arr = jnp.asarray(d, jnp.float32).reshape(1)
    start_arr = jnp.clip(
        jnp.asarray(budget, jnp.int32) - _SEL, 0, _PER - _SEL).reshape(1)
    out = _tc_main(g, par, jnp.asarray(_W), ids3, d_arr, start_arr)
    return out.reshape(_N_CLS * _SEL)


# speedup vs baseline: 20.8798x; 1.4269x over previous
import functools

import numpy as np
import jax
import jax.numpy as jnp
from jax import lax
from jax.experimental import pallas as pl
from jax.experimental.pallas import tpu as pltpu
from jax.experimental.pallas import tpu_sc as plsc

_N_CLS = 20
_PER = 200
_SEL = 100
_DIM = 64
_NPAD = 4096
_NC = 2
_NS = 16
_NW = _NC * _NS
_BPW = _NPAD // _NW


_M32 = np.uint64(0xFFFFFFFF)


def _tf2x32(k0, k1, x0, x1):
    k0 = np.asarray(k0).astype(np.uint64)
    k1 = np.asarray(k1).astype(np.uint64)
    x0 = np.asarray(x0).astype(np.uint64)
    x1 = np.asarray(x1).astype(np.uint64)
    ks2 = (k0 ^ k1 ^ np.uint64(0x1BD11BDA)) & _M32
    ks = (k0, k1, ks2)
    rots = ((13, 15, 26, 6), (17, 29, 16, 24))
    x0 = (x0 + k0) & _M32
    x1 = (x1 + k1) & _M32
    for i in range(5):
        for r in rots[i % 2]:
            x0 = (x0 + x1) & _M32
            x1 = (((x1 << np.uint64(r)) | (x1 >> np.uint64(32 - r))) & _M32) ^ x0
        x0 = (x0 + ks[(i + 1) % 3]) & _M32
        x1 = (x1 + ks[(i + 2) % 3] + np.uint64(i + 1)) & _M32
    return x0.astype(np.uint32), x1.astype(np.uint32)


def _np_randint(key, n, span):
    b1, b2 = _tf2x32(key[0], key[1], np.zeros(2, np.uint32),
                     np.arange(2, dtype=np.uint32))
    k1, k2 = (b1[0], b2[0]), (b1[1], b2[1])
    zeros = np.zeros(n, np.uint32)
    cnt = np.arange(n, dtype=np.uint32)
    hi = (lambda a, b: a ^ b)(*_tf2x32(k1[0], k1[1], zeros, cnt)).astype(np.uint64)
    lo = (lambda a, b: a ^ b)(*_tf2x32(k2[0], k2[1], zeros, cnt)).astype(np.uint64)
    span = np.uint64(span)
    mult = np.uint64(2 ** 16) % span
    mult = (mult * mult) % span
    return (((hi % span) * mult + (lo % span)) % span).astype(np.int64)


def _build_weights() -> np.ndarray:
    w = np.zeros((_N_CLS, _NPAD), np.float32)
    for i in range(_N_CLS):
        for j in range(_N_CLS):
            if j == i:
                continue
            fk0, fk1 = _tf2x32(0, 1, 0, i * _N_CLS + j)
            pick = _np_randint((fk0, fk1), _PER, _PER)
            np.add.at(w[i], j * _PER + pick, 1.0)
    return w.reshape(_N_CLS, 1, _NPAD)


_W = _build_weights()


def _sc_gather(table, idx):
    mesh = plsc.ScalarSubcoreMesh(axis_name="c", num_cores=_NC)
    half = _NPAD // _NC
    chunk = 512

    @functools.partial(
        pl.kernel,
        mesh=mesh,
        out_type=jax.ShapeDtypeStruct((_NPAD, _DIM), jnp.float32),
        scratch_types=[
            pltpu.SMEM((chunk,), jnp.int32),
            pltpu.SemaphoreType.DMA,
            pltpu.SemaphoreType.DMA,
        ],
    )
    def gk(table_hbm, idx_hbm, out_hbm, idx_s, isem, sem):
        base = lax.axis_index("c") * half

        def do_chunk(c, carry):
            cbase = base + c * chunk
            pltpu.async_copy(idx_hbm.at[pl.ds(cbase, chunk)], idx_s,
                             isem).wait()

            def issue(j, carry2):
                pltpu.async_copy(table_hbm.at[pl.ds(idx_s[j], 1), :],
                                 out_hbm.at[pl.ds(cbase + j, 1), :], sem)
                return carry2

            lax.fori_loop(0, chunk, issue, 0)

            def drain(j, carry2):
                pltpu.make_async_copy(
                    table_hbm.at[pl.ds(0, 1), :],
                    out_hbm.at[pl.ds(cbase + j, 1), :], sem).wait()
                return carry2

            lax.fori_loop(0, chunk, drain, 0)
            return carry

        lax.fori_loop(0, half // chunk, do_chunk, 0)

    return gk(table, idx)


def _f16_quant(x):
    bits = lax.bitcast_convert_type(x, jnp.int32)
    absb = bits & 0x7FFFFFFF
    signb = bits ^ absb
    lsb = (absb >> 13) & 1
    rounded = (absb + 4095 + lsb) & jnp.int32(-8192)
    ynorm = lax.bitcast_convert_type(signb | rounded, jnp.float32)
    s = x * 16777216.0
    r = (s + 12582912.0) - 12582912.0
    ysub = r * 5.9604644775390625e-08
    return jnp.where(absb < (113 << 23), ysub, ynorm)


def _f16_lt_threshold(d32):
    db = lax.bitcast_convert_type(d32, jnp.int32)
    lsb = (db >> 13) & 1
    hb = ((db + 4095 + lsb) >> 13) - (112 << 10)
    pe, pm = hb >> 10, hb & 1023
    d16v = lax.bitcast_convert_type(((pe + 112) << 23) | (pm << 13),
                                    jnp.float32)
    pb = hb - 1
    qe, qm = pb >> 10, pb & 1023
    pred = lax.bitcast_convert_type(((qe + 112) << 23) | (qm << 13),
                                    jnp.float32)
    mid = 0.5 * (pred + d16v)
    midb = lax.bitcast_convert_type(mid, jnp.int32)
    return lax.bitcast_convert_type(midb + (hb & 1), jnp.float32)


def _row_of(col):
    return lax.dot_general(
        jnp.ones((1, 1), jnp.float32), col,
        (((1,), (1,)), ((), ())),
        precision=lax.Precision.HIGHEST,
        preferred_element_type=jnp.float32,
    )


def _tc_body(d_ref, start_ref, g_ref, w_ref, ids_ref, out_ref,
             g16_ref, nb_ref):
    i = pl.program_id(0)

    @pl.when(i == 0)
    def _():
        g16_ref[...] = _f16_quant(g_ref[...])
        g32_0 = g16_ref[...]
        nb_col = jnp.sum(g32_0 * g32_0, axis=1, keepdims=True)
        nb_ref[...] = _row_of(nb_col)

    g32 = g16_ref[...]
    a32 = g16_ref[pl.ds(i * _PER, _PER), :]
    prod = lax.dot_general(
        a32, g32, (((1,), (1,)), ((), ())),
        precision=lax.Precision.DEFAULT,
        preferred_element_type=jnp.float32,
    )
    na = jnp.sum(a32 * a32, axis=1, keepdims=True)
    sq = (na + nb_ref[...]) - 2.0 * prod
    dist = jnp.sqrt(jnp.maximum(sq, 0.0))
    cmpf = (dist < _f16_lt_threshold(d_ref[0])).astype(jnp.float32)
    wsum = jnp.sum(cmpf * w_ref[0], axis=1, keepdims=True)
    counts = wsum.astype(jnp.int32)

    idx_col = lax.broadcasted_iota(jnp.int32, (_PER, 1), 0)
    key_col = counts * 256 + idx_col
    key_row = _row_of(key_col.astype(jnp.float32)).astype(jnp.int32)
    less = key_row < key_col
    rank = jnp.sum(less.astype(jnp.int32), axis=1, keepdims=True)

    tgt = lax.broadcasted_iota(jnp.int32, (1, _SEL), 1) + start_ref[0]
    hit = rank == tgt
    ids_col = ids_ref[0]
    out_ref[0] = jnp.sum(jnp.where(hit, ids_col, 0), axis=0, keepdims=True)


def _tc_main(g, w, ids3, d_arr, start_arr):
    return pl.pallas_call(
        _tc_body,
        grid=(_N_CLS,),
        in_specs=[
            pl.BlockSpec(memory_space=pltpu.SMEM),
            pl.BlockSpec(memory_space=pltpu.SMEM),
            pl.BlockSpec((_NPAD, _DIM), lambda i: (0, 0)),
            pl.BlockSpec((1, 1, _NPAD), lambda i: (i, 0, 0)),
            pl.BlockSpec((1, _PER, 1), lambda i: (i, 0, 0)),
        ],
        out_specs=pl.BlockSpec((1, 1, _SEL), lambda i: (i, 0, 0)),
        out_shape=jax.ShapeDtypeStruct((_N_CLS, 1, _SEL), jnp.int32),
        scratch_shapes=[
            pltpu.VMEM((_NPAD, _DIM), jnp.float32),
            pltpu.VMEM((1, _NPAD), jnp.float32),
        ],
    )(d_arr, start_arr, g, w, ids3)


def kernel(ids_per_cls_train, budget, feats, reps, d):
    ids_flat = ids_per_cls_train.reshape(-1).astype(jnp.int32)
    idx = jnp.concatenate(
        [ids_flat, jnp.zeros((_NPAD - _N_CLS * _PER,), jnp.int32)])
    g = _sc_gather(reps, idx)
    ids3 = ids_per_cls_train.reshape(_N_CLS, _PER, 1).astype(jnp.int32)
    d_arr = jnp.asarray(d, jnp.float32).reshape(1)
    start_arr = jnp.clip(
        jnp.asarray(budget, jnp.int32) - _SEL, 0, _PER - _SEL).reshape(1)
    out = _tc_main(g, jnp.asarray(_W), ids3, d_arr, start_arr)
    return out.reshape(_N_CLS * _SEL)


# speedup vs baseline: 20.8823x; 1.0001x over previous
import functools

import numpy as np
import jax
import jax.numpy as jnp
from jax import lax
from jax.experimental import pallas as pl
from jax.experimental.pallas import tpu as pltpu
from jax.experimental.pallas import tpu_sc as plsc

_N_CLS = 20
_PER = 200
_SEL = 100
_DIM = 64
_NPAD = 4096
_NC = 2
_NS = 16
_NW = _NC * _NS
_BPW = _NPAD // _NW


_M32 = np.uint64(0xFFFFFFFF)


def _tf2x32(k0, k1, x0, x1):
    k0 = np.asarray(k0).astype(np.uint64)
    k1 = np.asarray(k1).astype(np.uint64)
    x0 = np.asarray(x0).astype(np.uint64)
    x1 = np.asarray(x1).astype(np.uint64)
    ks2 = (k0 ^ k1 ^ np.uint64(0x1BD11BDA)) & _M32
    ks = (k0, k1, ks2)
    rots = ((13, 15, 26, 6), (17, 29, 16, 24))
    x0 = (x0 + k0) & _M32
    x1 = (x1 + k1) & _M32
    for i in range(5):
        for r in rots[i % 2]:
            x0 = (x0 + x1) & _M32
            x1 = (((x1 << np.uint64(r)) | (x1 >> np.uint64(32 - r))) & _M32) ^ x0
        x0 = (x0 + ks[(i + 1) % 3]) & _M32
        x1 = (x1 + ks[(i + 2) % 3] + np.uint64(i + 1)) & _M32
    return x0.astype(np.uint32), x1.astype(np.uint32)


def _np_randint(key, n, span):
    b1, b2 = _tf2x32(key[0], key[1], np.zeros(2, np.uint32),
                     np.arange(2, dtype=np.uint32))
    k1, k2 = (b1[0], b2[0]), (b1[1], b2[1])
    zeros = np.zeros(n, np.uint32)
    cnt = np.arange(n, dtype=np.uint32)
    hi = (lambda a, b: a ^ b)(*_tf2x32(k1[0], k1[1], zeros, cnt)).astype(np.uint64)
    lo = (lambda a, b: a ^ b)(*_tf2x32(k2[0], k2[1], zeros, cnt)).astype(np.uint64)
    span = np.uint64(span)
    mult = np.uint64(2 ** 16) % span
    mult = (mult * mult) % span
    return (((hi % span) * mult + (lo % span)) % span).astype(np.int64)


def _build_weights() -> np.ndarray:
    w = np.zeros((_N_CLS, _NPAD), np.float32)
    for i in range(_N_CLS):
        for j in range(_N_CLS):
            if j == i:
                continue
            fk0, fk1 = _tf2x32(0, 1, 0, i * _N_CLS + j)
            pick = _np_randint((fk0, fk1), _PER, _PER)
            np.add.at(w[i], j * _PER + pick, 1.0)
    return w.reshape(_N_CLS, 1, _NPAD)


_W = _build_weights()


def _sc_gather(table, idx):
    mesh = plsc.ScalarSubcoreMesh(axis_name="c", num_cores=_NC)
    half = _NPAD // _NC
    chunk = 512

    @functools.partial(
        pl.kernel,
        mesh=mesh,
        out_type=jax.ShapeDtypeStruct((_NPAD, _DIM), jnp.float32),
        scratch_types=[
            pltpu.SMEM((chunk,), jnp.int32),
            pltpu.SemaphoreType.DMA,
            pltpu.SemaphoreType.DMA,
        ],
        compiler_params=pltpu.CompilerParams(use_tc_tiling_on_sc=True),
    )
    def gk(table_hbm, idx_hbm, out_hbm, idx_s, isem, sem):
        base = lax.axis_index("c") * half

        def do_chunk(c, carry):
            cbase = base + c * chunk
            pltpu.async_copy(idx_hbm.at[pl.ds(cbase, chunk)], idx_s,
                             isem).wait()

            def issue(j, carry2):
                pltpu.async_copy(table_hbm.at[pl.ds(idx_s[j], 1), :],
                                 out_hbm.at[pl.ds(cbase + j, 1), :], sem)
                return carry2

            lax.fori_loop(0, chunk, issue, 0)

            def drain(j, carry2):
                pltpu.make_async_copy(
                    table_hbm.at[pl.ds(0, 1), :],
                    out_hbm.at[pl.ds(cbase + j, 1), :], sem).wait()
                return carry2

            lax.fori_loop(0, chunk, drain, 0)
            return carry

        lax.fori_loop(0, half // chunk, do_chunk, 0)

    return gk(table, idx)


def _f16_quant(x):
    bits = lax.bitcast_convert_type(x, jnp.int32)
    absb = bits & 0x7FFFFFFF
    signb = bits ^ absb
    lsb = (absb >> 13) & 1
    rounded = (absb + 4095 + lsb) & jnp.int32(-8192)
    ynorm = lax.bitcast_convert_type(signb | rounded, jnp.float32)
    s = x * 16777216.0
    r = (s + 12582912.0) - 12582912.0
    ysub = r * 5.9604644775390625e-08
    return jnp.where(absb < (113 << 23), ysub, ynorm)


def _f16_lt_threshold(d32):
    db = lax.bitcast_convert_type(d32, jnp.int32)
    lsb = (db >> 13) & 1
    hb = ((db + 4095 + lsb) >> 13) - (112 << 10)
    pe, pm = hb >> 10, hb & 1023
    d16v = lax.bitcast_convert_type(((pe + 112) << 23) | (pm << 13),
                                    jnp.float32)
    pb = hb - 1
    qe, qm = pb >> 10, pb & 1023
    pred = lax.bitcast_convert_type(((qe + 112) << 23) | (qm << 13),
                                    jnp.float32)
    mid = 0.5 * (pred + d16v)
    midb = lax.bitcast_convert_type(mid, jnp.int32)
    return lax.bitcast_convert_type(midb + (hb & 1), jnp.float32)


def _row_of(col):
    return lax.dot_general(
        jnp.ones((1, 1), jnp.float32), col,
        (((1,), (1,)), ((), ())),
        precision=lax.Precision.HIGHEST,
        preferred_element_type=jnp.float32,
    )


def _tc_body(d_ref, start_ref, g_ref, w_ref, ids_ref, out_ref,
             g16_ref, nb_ref):
    i = pl.program_id(0)

    @pl.when(i == 0)
    def _():
        g16_ref[...] = _f16_quant(g_ref[...])
        g32_0 = g16_ref[...]
        nb_col = jnp.sum(g32_0 * g32_0, axis=1, keepdims=True)
        nb_ref[...] = _row_of(nb_col)

    g32 = g16_ref[...]
    a32 = g16_ref[pl.ds(i * _PER, _PER), :]
    prod = lax.dot_general(
        a32, g32, (((1,), (1,)), ((), ())),
        precision=lax.Precision.DEFAULT,
        preferred_element_type=jnp.float32,
    )
    na = jnp.sum(a32 * a32, axis=1, keepdims=True)
    sq = (na + nb_ref[...]) - 2.0 * prod
    dist = jnp.sqrt(jnp.maximum(sq, 0.0))
    cmpf = (dist < _f16_lt_threshold(d_ref[0])).astype(jnp.float32)
    wsum = jnp.sum(cmpf * w_ref[0], axis=1, keepdims=True)
    counts = wsum.astype(jnp.int32)

    idx_col = lax.broadcasted_iota(jnp.int32, (_PER, 1), 0)
    key_col = counts * 256 + idx_col
    key_row = _row_of(key_col.astype(jnp.float32)).astype(jnp.int32)
    less = key_row < key_col
    rank = jnp.sum(less.astype(jnp.int32), axis=1, keepdims=True)

    tgt = lax.broadcasted_iota(jnp.int32, (1, _SEL), 1) + start_ref[0]
    hit = rank == tgt
    ids_col = ids_ref[0]
    out_ref[0] = jnp.sum(jnp.where(hit, ids_col, 0), axis=0, keepdims=True)


def _tc_main(g, w, ids3, d_arr, start_arr):
    return pl.pallas_call(
        _tc_body,
        grid=(_N_CLS,),
        in_specs=[
            pl.BlockSpec(memory_space=pltpu.SMEM),
            pl.BlockSpec(memory_space=pltpu.SMEM),
            pl.BlockSpec((_NPAD, _DIM), lambda i: (0, 0)),
            pl.BlockSpec((1, 1, _NPAD), lambda i: (i, 0, 0)),
            pl.BlockSpec((1, _PER, 1), lambda i: (i, 0, 0)),
        ],
        out_specs=pl.BlockSpec((1, 1, _SEL), lambda i: (i, 0, 0)),
        out_shape=jax.ShapeDtypeStruct((_N_CLS, 1, _SEL), jnp.int32),
        scratch_shapes=[
            pltpu.VMEM((_NPAD, _DIM), jnp.float32),
            pltpu.VMEM((1, _NPAD), jnp.float32),
        ],
    )(d_arr, start_arr, g, w, ids3)


def kernel(ids_per_cls_train, budget, feats, reps, d):
    ids_flat = ids_per_cls_train.reshape(-1).astype(jnp.int32)
    idx = jnp.concatenate(
        [ids_flat, jnp.zeros((_NPAD - _N_CLS * _PER,), jnp.int32)])
    g = _sc_gather(reps, idx)
    ids3 = ids_per_cls_train.reshape(_N_CLS, _PER, 1).astype(jnp.int32)
    d_arr = jnp.asarray(d, jnp.float32).reshape(1)
    start_arr = jnp.clip(
        jnp.asarray(budget, jnp.int32) - _SEL, 0, _PER - _SEL).reshape(1)
    out = _tc_main(g, jnp.asarray(_W), ids3, d_arr, start_arr)
    return out.reshape(_N_CLS * _SEL)


# speedup vs baseline: 22.2294x; 1.0645x over previous
import functools

import numpy as np
import jax
import jax.numpy as jnp
from jax import lax
from jax.experimental import pallas as pl
from jax.experimental.pallas import tpu as pltpu
from jax.experimental.pallas import tpu_sc as plsc

_N_CLS = 20
_PER = 200
_SEL = 100
_DIM = 64
_NPAD = 4096
_NC = 2
_NS = 16
_NW = _NC * _NS
_BPW = _NPAD // _NW


_M32 = np.uint64(0xFFFFFFFF)


def _tf2x32(k0, k1, x0, x1):
    k0 = np.asarray(k0).astype(np.uint64)
    k1 = np.asarray(k1).astype(np.uint64)
    x0 = np.asarray(x0).astype(np.uint64)
    x1 = np.asarray(x1).astype(np.uint64)
    ks2 = (k0 ^ k1 ^ np.uint64(0x1BD11BDA)) & _M32
    ks = (k0, k1, ks2)
    rots = ((13, 15, 26, 6), (17, 29, 16, 24))
    x0 = (x0 + k0) & _M32
    x1 = (x1 + k1) & _M32
    for i in range(5):
        for r in rots[i % 2]:
            x0 = (x0 + x1) & _M32
            x1 = (((x1 << np.uint64(r)) | (x1 >> np.uint64(32 - r))) & _M32) ^ x0
        x0 = (x0 + ks[(i + 1) % 3]) & _M32
        x1 = (x1 + ks[(i + 2) % 3] + np.uint64(i + 1)) & _M32
    return x0.astype(np.uint32), x1.astype(np.uint32)


def _np_randint(key, n, span):
    b1, b2 = _tf2x32(key[0], key[1], np.zeros(2, np.uint32),
                     np.arange(2, dtype=np.uint32))
    k1, k2 = (b1[0], b2[0]), (b1[1], b2[1])
    zeros = np.zeros(n, np.uint32)
    cnt = np.arange(n, dtype=np.uint32)
    hi = (lambda a, b: a ^ b)(*_tf2x32(k1[0], k1[1], zeros, cnt)).astype(np.uint64)
    lo = (lambda a, b: a ^ b)(*_tf2x32(k2[0], k2[1], zeros, cnt)).astype(np.uint64)
    span = np.uint64(span)
    mult = np.uint64(2 ** 16) % span
    mult = (mult * mult) % span
    return (((hi % span) * mult + (lo % span)) % span).astype(np.int64)


def _build_weights() -> np.ndarray:
    w = np.zeros((_N_CLS, _NPAD), np.float32)
    for i in range(_N_CLS):
        for j in range(_N_CLS):
            if j == i:
                continue
            fk0, fk1 = _tf2x32(0, 1, 0, i * _N_CLS + j)
            pick = _np_randint((fk0, fk1), _PER, _PER)
            np.add.at(w[i], j * _PER + pick, 1.0)
    return w.reshape(_N_CLS, 1, _NPAD)


_W = _build_weights()


def _sc_gather(table, idx):
    mesh = plsc.ScalarSubcoreMesh(axis_name="c", num_cores=_NC)
    half = _NPAD // _NC
    chunk = 512

    @functools.partial(
        pl.kernel,
        mesh=mesh,
        out_type=jax.ShapeDtypeStruct((_NPAD, _DIM), jnp.float32),
        scratch_types=[
            pltpu.SMEM((chunk,), jnp.int32),
            pltpu.SemaphoreType.DMA,
            pltpu.SemaphoreType.DMA,
        ],
    )
    def gk(table_hbm, idx_hbm, out_hbm, idx_s, isem, sem):
        base = lax.axis_index("c") * half

        def do_chunk(c, carry):
            cbase = base + c * chunk
            pltpu.async_copy(idx_hbm.at[pl.ds(cbase, chunk)], idx_s,
                             isem).wait()

            def issue(j, carry2):
                pltpu.async_copy(table_hbm.at[pl.ds(idx_s[j], 1), :],
                                 out_hbm.at[pl.ds(cbase + j, 1), :], sem)
                return carry2

            lax.fori_loop(0, chunk, issue, 0)

            def drain(j, carry2):
                pltpu.make_async_copy(
                    table_hbm.at[pl.ds(0, 1), :],
                    out_hbm.at[pl.ds(cbase + j, 1), :], sem).wait()
                return carry2

            lax.fori_loop(0, chunk, drain, 0)
            return carry

        lax.fori_loop(0, half // chunk, do_chunk, 0)

    return gk(table, idx)


def _f16_quant(x):
    bits = lax.bitcast_convert_type(x, jnp.int32)
    absb = bits & 0x7FFFFFFF
    signb = bits ^ absb
    lsb = (absb >> 13) & 1
    rounded = (absb + 4095 + lsb) & jnp.int32(-8192)
    ynorm = lax.bitcast_convert_type(signb | rounded, jnp.float32)
    s = x * 16777216.0
    r = (s + 12582912.0) - 12582912.0
    ysub = r * 5.9604644775390625e-08
    return jnp.where(absb < (113 << 23), ysub, ynorm)


def _f16_lt_threshold(d32):
    db = lax.bitcast_convert_type(d32, jnp.int32)
    lsb = (db >> 13) & 1
    hb = ((db + 4095 + lsb) >> 13) - (112 << 10)
    pe, pm = hb >> 10, hb & 1023
    d16v = lax.bitcast_convert_type(((pe + 112) << 23) | (pm << 13),
                                    jnp.float32)
    pb = hb - 1
    qe, qm = pb >> 10, pb & 1023
    pred = lax.bitcast_convert_type(((qe + 112) << 23) | (qm << 13),
                                    jnp.float32)
    mid = 0.5 * (pred + d16v)
    midb = lax.bitcast_convert_type(mid, jnp.int32)
    return lax.bitcast_convert_type(midb + (hb & 1), jnp.float32)


def _row_of(col):
    return lax.dot_general(
        jnp.ones((1, 1), jnp.float32), col,
        (((1,), (1,)), ((), ())),
        precision=lax.Precision.HIGHEST,
        preferred_element_type=jnp.float32,
    )


def _tc_body(d_ref, start_ref, idx_ref, table_ref, w_ref, ids_ref, out_ref,
             graw_ref, g16_ref, nb_ref, sem):
    i = pl.program_id(0)

    @pl.when(i == 0)
    def _():
        def issue(j, carry):
            pltpu.make_async_copy(table_ref.at[pl.ds(idx_ref[j], 1), :],
                                  graw_ref.at[pl.ds(j, 1), :], sem).start()
            return carry

        lax.fori_loop(0, _NPAD, issue, 0)

        def drain(j, carry):
            pltpu.make_async_copy(table_ref.at[pl.ds(0, 1), :],
                                  graw_ref.at[pl.ds(j, 1), :], sem).wait()
            return carry

        lax.fori_loop(0, _NPAD, drain, 0)

        g16_ref[...] = _f16_quant(graw_ref[...])
        g32_0 = g16_ref[...]
        nb_col = jnp.sum(g32_0 * g32_0, axis=1, keepdims=True)
        nb_ref[...] = _row_of(nb_col)

    g32 = g16_ref[...]
    a32 = g16_ref[pl.ds(i * _PER, _PER), :]
    prod = lax.dot_general(
        a32, g32, (((1,), (1,)), ((), ())),
        precision=lax.Precision.DEFAULT,
        preferred_element_type=jnp.float32,
    )
    na = jnp.sum(a32 * a32, axis=1, keepdims=True)
    sq = (na + nb_ref[...]) - 2.0 * prod
    dist = jnp.sqrt(jnp.maximum(sq, 0.0))
    cmpf = (dist < _f16_lt_threshold(d_ref[0])).astype(jnp.float32)
    wsum = jnp.sum(cmpf * w_ref[0], axis=1, keepdims=True)
    counts = wsum.astype(jnp.int32)

    idx_col = lax.broadcasted_iota(jnp.int32, (_PER, 1), 0)
    key_col = counts * 256 + idx_col
    key_row = _row_of(key_col.astype(jnp.float32)).astype(jnp.int32)
    less = key_row < key_col
    rank = jnp.sum(less.astype(jnp.int32), axis=1, keepdims=True)

    tgt = lax.broadcasted_iota(jnp.int32, (1, _SEL), 1) + start_ref[0]
    hit = rank == tgt
    ids_col = ids_ref[0]
    out_ref[0] = jnp.sum(jnp.where(hit, ids_col, 0), axis=0, keepdims=True)


def _tc_main(table, idx, w, ids3, d_arr, start_arr):
    return pl.pallas_call(
        _tc_body,
        grid=(_N_CLS,),
        in_specs=[
            pl.BlockSpec(memory_space=pltpu.SMEM),
            pl.BlockSpec(memory_space=pltpu.SMEM),
            pl.BlockSpec(memory_space=pltpu.SMEM),
            pl.BlockSpec(memory_space=pltpu.MemorySpace.HBM),
            pl.BlockSpec((1, 1, _NPAD), lambda i: (i, 0, 0)),
            pl.BlockSpec((1, _PER, 1), lambda i: (i, 0, 0)),
        ],
        out_specs=pl.BlockSpec((1, 1, _SEL), lambda i: (i, 0, 0)),
        out_shape=jax.ShapeDtypeStruct((_N_CLS, 1, _SEL), jnp.int32),
        scratch_shapes=[
            pltpu.VMEM((_NPAD, _DIM), jnp.float32),
            pltpu.VMEM((_NPAD, _DIM), jnp.float32),
            pltpu.VMEM((1, _NPAD), jnp.float32),
            pltpu.SemaphoreType.DMA,
        ],
    )(d_arr, start_arr, idx, table, w, ids3)


def kernel(ids_per_cls_train, budget, feats, reps, d):
    ids_flat = ids_per_cls_train.reshape(-1).astype(jnp.int32)
    idx = jnp.concatenate(
        [ids_flat, jnp.zeros((_NPAD - _N_CLS * _PER,), jnp.int32)])
    ids3 = ids_per_cls_train.reshape(_N_CLS, _PER, 1).astype(jnp.int32)
    d_arr = jnp.asarray(d, jnp.float32).reshape(1)
    start_arr = jnp.clip(
        jnp.asarray(budget, jnp.int32) - _SEL, 0, _PER - _SEL).reshape(1)
    out = _tc_main(reps, idx, jnp.asarray(_W), ids3, d_arr, start_arr)
    return out.reshape(_N_CLS * _SEL)
